# Initial kernel scaffold; baseline (speedup 1.0000x reference)
import jax
import jax.numpy as jnp
from jax.experimental import pallas as pl


def kernel(y_author, y_paper, y_venue, edge_author_paper, edge_paper_author, edge_paper_venue, edge_venue_paper):
    raise NotImplementedError("write your pallas kernel here")


# speedup vs baseline: 6.1921x; 6.1921x over previous
import functools

import jax
import jax.numpy as jnp
from jax import lax
from jax.experimental import pallas as pl
from jax.experimental.pallas import tpu as pltpu
from jax.experimental.pallas import tpu_sc as plsc

N = 10000
D = 128
E = 320000

NC = 2
NS = 16
HD = D // NC
K = 128
CHUNKS = -(-E // (NS * K))
EPT = CHUNKS * K
E_PAD = EPT * NS
DEG_W = 16
N_PAD = 10112
RPT = N_PAD // NS


@functools.partial(
    pl.kernel,
    mesh=plsc.VectorSubcoreMesh(core_axis_name="c", subcore_axis_name="s"),
    compiler_params=pltpu.CompilerParams(use_tc_tiling_on_sc=False),
    out_type=[
        jax.ShapeDtypeStruct((NC, N_PAD, HD), jnp.float32),
        jax.ShapeDtypeStruct((N_PAD, DEG_W), jnp.float32),
    ],
    scratch_types=[
        pltpu.VMEM((CHUNKS, K), jnp.int32),
        pltpu.VMEM((CHUNKS, K), jnp.int32),
        pltpu.VMEM((K, HD), jnp.float32),
        pltpu.VMEM((K, DEG_W), jnp.float32),
        pltpu.VMEM_SHARED((N_PAD, HD), jnp.float32),
        pltpu.VMEM_SHARED((N_PAD, DEG_W), jnp.float32),
        pltpu.SemaphoreType.DMA,
    ],
)
def _sc_propagate(y_hbm, src_hbm, dst_hbm, zacc_hbm, zdeg_hbm,
                  acc_out, deg_out,
                  src_v, dst_v, rows_v, ones_v, acc_sh, deg_sh, sem):
    cid = lax.axis_index("c")
    sid = lax.axis_index("s")

    pltpu.sync_copy(zacc_hbm, acc_sh.at[pl.ds(sid * RPT, RPT)])
    pltpu.sync_copy(zdeg_hbm, deg_sh.at[pl.ds(sid * RPT, RPT)])

    def _set_ones(i, carry):
        ones_v[i, :] = jnp.ones((DEG_W,), jnp.float32)
        return carry
    lax.fori_loop(0, K, _set_ones, 0)

    pltpu.sync_copy(src_hbm.at[cid, sid], src_v)
    pltpu.sync_copy(dst_hbm.at[sid], dst_v)

    plsc.subcore_barrier()

    def _chunk(j, carry):
        pltpu.async_copy(y_hbm.at[src_v.at[j]], rows_v, sem).wait()
        pltpu.sync_copy(rows_v, acc_sh.at[dst_v.at[j]], add=True)

        @pl.when(cid == 0)
        def _():
            pltpu.sync_copy(ones_v, deg_sh.at[dst_v.at[j]], add=True)
        return carry
    lax.fori_loop(0, CHUNKS, _chunk, 0)

    plsc.subcore_barrier()

    pltpu.sync_copy(acc_sh.at[pl.ds(sid * RPT, RPT)],
                    acc_out.at[cid, pl.ds(sid * RPT, RPT)])

    @pl.when(cid == 0)
    def _():
        pltpu.sync_copy(deg_sh.at[pl.ds(sid * RPT, RPT)],
                        deg_out.at[pl.ds(sid * RPT, RPT)])


_BLK = 400


def _combine_body(acc_ref, deg_ref, out_ref):
    d = jnp.maximum(deg_ref[:, 0:1], 1.0)
    out_ref[:, 0:HD] = acc_ref[0] / d
    out_ref[:, HD:D] = acc_ref[1] / d


def _combine(acc_p, deg_p):
    return pl.pallas_call(
        _combine_body,
        grid=(N // _BLK,),
        in_specs=[
            pl.BlockSpec((NC, _BLK, HD), lambda i: (0, i, 0)),
            pl.BlockSpec((_BLK, DEG_W), lambda i: (i, 0)),
        ],
        out_specs=pl.BlockSpec((_BLK, D), lambda i: (i, 0)),
        out_shape=jax.ShapeDtypeStruct((N, D), jnp.float32),
    )(acc_p, deg_p)


@jax.jit
def kernel(y_author, y_paper, y_venue, edge_author_paper, edge_paper_author,
           edge_paper_venue, edge_venue_paper):
    src = edge_paper_author[0].astype(jnp.int32)
    dst = edge_paper_author[1].astype(jnp.int32)
    pad = E_PAD - E
    src = jnp.concatenate([src, jnp.zeros((pad,), jnp.int32)])
    dst = jnp.concatenate([dst, jnp.full((pad,), N, jnp.int32)])
    src2 = jnp.stack([src, src + N]).reshape(NC, NS, CHUNKS, K)
    dst = dst.reshape(NS, CHUNKS, K)
    y_stack = jnp.concatenate(
        [y_paper[:, :HD], y_paper[:, HD:]], axis=0)
    zacc = jnp.zeros((RPT, HD), jnp.float32)
    zdeg = jnp.zeros((RPT, DEG_W), jnp.float32)
    acc_p, deg_p = _sc_propagate(y_stack, src2, dst, zacc, zdeg)
    return _combine(acc_p[:, :N], deg_p[:N])


# speedup vs baseline: 6.7880x; 1.0962x over previous
import functools

import jax
import jax.numpy as jnp
from jax import lax
from jax.experimental import pallas as pl
from jax.experimental.pallas import tpu as pltpu
from jax.experimental.pallas import tpu_sc as plsc

N = 10000
D = 128
E = 320000

NC = 2
NS = 16
HD = D // NC
K = 128
CHUNKS = 158
EPT = CHUNKS * K
E_PAD = EPT * NS
HC = CHUNKS // 2
DEG_W = 16
N_PAD = 10112
RPT = N_PAD // NS


@functools.partial(
    pl.kernel,
    mesh=plsc.VectorSubcoreMesh(core_axis_name="c", subcore_axis_name="s"),
    compiler_params=pltpu.CompilerParams(use_tc_tiling_on_sc=False),
    out_type=[
        jax.ShapeDtypeStruct((NC, N_PAD, HD), jnp.float32),
        jax.ShapeDtypeStruct((N_PAD, DEG_W), jnp.float32),
    ],
    scratch_types=[
        pltpu.VMEM((CHUNKS, K), jnp.int32),
        pltpu.VMEM((CHUNKS, K), jnp.int32),
        pltpu.VMEM((2, K, HD), jnp.float32),
        pltpu.VMEM((K, DEG_W), jnp.float32),
        pltpu.VMEM_SHARED((N_PAD, HD), jnp.float32),
        pltpu.VMEM_SHARED((N_PAD, DEG_W), jnp.float32),
        pltpu.SemaphoreType.DMA,
        pltpu.SemaphoreType.DMA,
        pltpu.SemaphoreType.DMA,
        pltpu.SemaphoreType.DMA,
        pltpu.SemaphoreType.DMA,
        pltpu.SemaphoreType.DMA,
    ],
)
def _sc_propagate(y_hbm, src_hbm, dst_hbm, zacc_hbm, zdeg_hbm,
                  acc_out, deg_out,
                  src_v, dst_v, rows_v, ones_v, acc_sh, deg_sh,
                  sg0, sg1, ss0, ss1, sd0, sd1):
    cid = lax.axis_index("c")
    sid = lax.axis_index("s")
    sg = (sg0, sg1)
    ss = (ss0, ss1)
    sd = (sd0, sd1)

    pltpu.sync_copy(zacc_hbm, acc_sh.at[pl.ds(sid * RPT, RPT)])
    pltpu.sync_copy(zdeg_hbm, deg_sh.at[pl.ds(sid * RPT, RPT)])

    def _set_ones(i, carry):
        ones_v[i, :] = jnp.ones((DEG_W,), jnp.float32)
        return carry
    lax.fori_loop(0, K, _set_ones, 0)

    pltpu.sync_copy(src_hbm.at[cid, sid], src_v)
    pltpu.sync_copy(dst_hbm.at[sid], dst_v)

    plsc.subcore_barrier()

    def _gather(j, b):
        return pltpu.make_async_copy(
            y_hbm.at[src_v.at[j]], rows_v.at[b], sg[b])

    def _scatter(j, b):
        return pltpu.make_async_copy(
            rows_v.at[b], acc_sh.at[dst_v.at[j]], ss[b])

    def _degree(j, b):
        return pltpu.make_async_copy(
            ones_v, deg_sh.at[dst_v.at[j]], sd[b])

    _gather(0, 0).start()

    def _pair(jo, carry):
        for b in (0, 1):
            j = 2 * jo + b
            _gather(j, b).wait()
            _scatter(j, b).start(add=True)

            @pl.when(cid == 0)
            def _():
                _degree(j, b).start(add=True)

            def _retire():
                _scatter(j - 1, 1 - b).wait()

                @pl.when(cid == 0)
                def _():
                    _degree(j - 1, 1 - b).wait()

            def _next_gather():
                _gather(j + 1, 1 - b).start()

            if b == 0:
                @pl.when(jo >= 1)
                def _():
                    _retire()
                _next_gather()
            else:
                _retire()

                @pl.when(jo < HC - 1)
                def _():
                    _next_gather()
        return carry
    lax.fori_loop(0, HC, _pair, 0)

    _scatter(CHUNKS - 1, 1).wait()

    @pl.when(cid == 0)
    def _():
        _degree(CHUNKS - 1, 1).wait()

    plsc.subcore_barrier()

    pltpu.sync_copy(acc_sh.at[pl.ds(sid * RPT, RPT)],
                    acc_out.at[cid, pl.ds(sid * RPT, RPT)])

    @pl.when(cid == 0)
    def _():
        pltpu.sync_copy(deg_sh.at[pl.ds(sid * RPT, RPT)],
                        deg_out.at[pl.ds(sid * RPT, RPT)])


_BLK = 400


def _combine_body(acc_ref, deg_ref, out_ref):
    d = jnp.maximum(deg_ref[:, 0:1], 1.0)
    out_ref[:, 0:HD] = acc_ref[0] / d
    out_ref[:, HD:D] = acc_ref[1] / d


def _combine(acc_p, deg_p):
    return pl.pallas_call(
        _combine_body,
        grid=(N // _BLK,),
        in_specs=[
            pl.BlockSpec((NC, _BLK, HD), lambda i: (0, i, 0)),
            pl.BlockSpec((_BLK, DEG_W), lambda i: (i, 0)),
        ],
        out_specs=pl.BlockSpec((_BLK, D), lambda i: (i, 0)),
        out_shape=jax.ShapeDtypeStruct((N, D), jnp.float32),
    )(acc_p, deg_p)


@jax.jit
def kernel(y_author, y_paper, y_venue, edge_author_paper, edge_paper_author,
           edge_paper_venue, edge_venue_paper):
    src = edge_paper_author[0].astype(jnp.int32)
    dst = edge_paper_author[1].astype(jnp.int32)
    pad = E_PAD - E
    src = jnp.concatenate([src, jnp.zeros((pad,), jnp.int32)])
    dst = jnp.concatenate([dst, jnp.full((pad,), N, jnp.int32)])
    src2 = jnp.stack([src, src + N]).reshape(NC, NS, CHUNKS, K)
    dst = dst.reshape(NS, CHUNKS, K)
    y_stack = jnp.concatenate(
        [y_paper[:, :HD], y_paper[:, HD:]], axis=0)
    zacc = jnp.zeros((RPT, HD), jnp.float32)
    zdeg = jnp.zeros((RPT, DEG_W), jnp.float32)
    acc_p, deg_p = _sc_propagate(y_stack, src2, dst, zacc, zdeg)
    return _combine(acc_p, deg_p)
